# baseline (speedup 1.0000x reference)
import functools
import math

import jax
import jax.numpy as jnp
from jax import lax
from jax.experimental import pallas as pl
from jax.experimental.pallas import tpu as pltpu
from jax.experimental.pallas import tpu_sc as plsc

_N = 16384
_K = 128
_D16 = 128
_NC = 2
_NS = 16
_NW = _NC * _NS
_CHUNK = _N // _NW
_NIDX = _CHUNK // 128

_BLK = 2048
_NB = _N // _BLK
_LT = _BLK // 128
_QV = float(math.atanh(0.5) ** 2 + 1.0)
_EPS_D = 1e-9

_sc_mesh = plsc.VectorSubcoreMesh(core_axis_name="c", subcore_axis_name="s")


@functools.partial(
    pl.kernel,
    mesh=_sc_mesh,
    out_type=jax.ShapeDtypeStruct((_NC, _K, _D16), jnp.float32),
    scratch_types=[
        pltpu.VMEM((128,), jnp.int32),
        pltpu.VMEM((128,), jnp.int32),
        pltpu.VMEM((128,), jnp.int32),
        pltpu.VMEM((128,), jnp.int32),
        pltpu.VMEM((128, _D16), jnp.float32),
        pltpu.VMEM((128, _D16), jnp.float32),
        pltpu.VMEM((128, _D16), jnp.float32),
        pltpu.VMEM((128, _D16), jnp.float32),
        pltpu.VMEM((_K, _D16), jnp.float32),
        pltpu.VMEM_SHARED((_K, _D16), jnp.float32),
    ],
)
def _segsum_sc(data_hbm, tidx_hbm, out_hbm,
               idx_v0, idx_v1, idx_v2, idx_v3,
               stage_v0, stage_v1, stage_v2, stage_v3, zero_v, shared):
    cid = lax.axis_index("c")
    sid = lax.axis_index("s")
    wid = sid * _NC + cid
    base = wid * _CHUNK

    @pl.when(sid == 0)
    def _init():
        z = jnp.zeros((_D16,), jnp.float32)
        for i in range(_K):
            zero_v[i, :] = z
        pltpu.sync_copy(zero_v, shared)

    plsc.subcore_barrier()
    idx_refs = (idx_v0, idx_v1, idx_v2, idx_v3)
    stage_refs = (stage_v0, stage_v1, stage_v2, stage_v3)
    for j in range(_NIDX):
        pltpu.sync_copy(data_hbm.at[pl.ds(base + j * 128, 128)], stage_refs[j])
        pltpu.sync_copy(tidx_hbm.at[wid, j], idx_refs[j])
    for j in range(_NIDX):
        pltpu.sync_copy(stage_refs[j], shared.at[idx_refs[j]], add=True)
    plsc.subcore_barrier()

    @pl.when(sid == 0)
    def _flush():
        pltpu.sync_copy(shared, out_hbm.at[cid])


def _tc_body(coords_t_ref, tidx_ref, segtab_ref, out_ref, bmat_ref, acc_ref):
    b = pl.program_id(0)

    @pl.when(b == 0)
    def _xk():
        acc_ref[...] = jnp.zeros_like(acc_ref)
        seg = segtab_ref[0] + segtab_ref[1]
        cnt = seg[:, 3:4]
        den = _QV / (_QV * cnt + 1e-9)
        xk = seg[:, 0:3] * den
        ck2e = (jnp.sum(xk * xk, axis=1, keepdims=True) + _EPS_D
                + jnp.where(cnt > 0.0, 0.0, 1e6))
        bmat_ref[...] = jnp.concatenate(
            [-2.0 * xk, jnp.ones((_K, 3), jnp.float32), ck2e,
             jnp.zeros((_K, 1), jnp.float32)], axis=1)

    t = tidx_ref[...]
    kio = jax.lax.broadcasted_iota(jnp.int32, (_K, _BLK), 0)
    mask = kio == t

    ct = coords_t_ref[...]
    a7 = jnp.concatenate(
        [ct, ct * ct, jnp.ones((1, _BLK), jnp.float32)], axis=0)
    d2e = jax.lax.dot_general(
        bmat_ref[:, 0:7], a7, (((1,), (0,)), ((), ())),
        preferred_element_type=jnp.float32)
    d2c = jnp.maximum(d2e, _EPS_D)
    dist = d2c * jax.lax.rsqrt(d2c)
    rep = jnp.maximum(0.0, 1.0 - dist)
    contrib = jnp.where(mask, d2c, rep)
    s = contrib[:, 0:128]
    for i in range(1, _LT):
        s = s + contrib[:, i * 128:(i + 1) * 128]
    acc_ref[...] += s

    @pl.when(b == _NB - 1)
    def _fin():
        out_ref[0, 0] = (_QV * _QV / (2.0 * _N)) * jnp.sum(acc_ref[...])


def kernel(coords, truth_indices, row_splits):
    del row_splits
    tidx = truth_indices.astype(jnp.int32)
    data16 = jnp.concatenate(
        [coords, jnp.ones((_N, 1), jnp.float32),
         jnp.zeros((_N, _D16 - 4), jnp.float32)], axis=1)
    tidx_sc = tidx.reshape(_NW, _NIDX, 128)
    segtab = _segsum_sc(data16, tidx_sc)

    coords_t = coords.T
    tidx_t = tidx.reshape(1, _N)
    out = pl.pallas_call(
        _tc_body,
        grid=(_NB,),
        in_specs=[
            pl.BlockSpec((3, _BLK), lambda b: (0, b)),
            pl.BlockSpec((1, _BLK), lambda b: (0, b)),
            pl.BlockSpec((_NC, _K, _D16), lambda b: (0, 0, 0)),
        ],
        out_specs=pl.BlockSpec((1, 1), lambda b: (0, 0), memory_space=pltpu.SMEM),
        out_shape=jax.ShapeDtypeStruct((1, 1), jnp.float32),
        scratch_shapes=[
            pltpu.VMEM((_K, 8), jnp.float32),
            pltpu.VMEM((_K, 128), jnp.float32),
        ],
    )(coords_t, tidx_t, segtab)
    return out[0, 0]


# speedup vs baseline: 1.2112x; 1.2112x over previous
import functools
import math

import jax
import jax.numpy as jnp
from jax import lax
from jax.experimental import pallas as pl
from jax.experimental.pallas import tpu as pltpu
from jax.experimental.pallas import tpu_sc as plsc

_N = 16384
_K = 128
_DW = 16
_NC = 2
_NS = 16
_NW = _NC * _NS
_CHUNK = _N // _NW

_BLK = 2048
_NB = _N // _BLK
_LT = _BLK // 128
_QV = float(math.atanh(0.5) ** 2 + 1.0)
_EPS_D = 1e-9

_sc_mesh = plsc.VectorSubcoreMesh(core_axis_name="c", subcore_axis_name="s")


@functools.partial(
    pl.kernel,
    mesh=_sc_mesh,
    out_type=jax.ShapeDtypeStruct((_NW, _K, _DW), jnp.float32),
    scratch_types=[
        pltpu.VMEM((_CHUNK, _DW), jnp.float32),
        pltpu.VMEM((_K, _DW), jnp.float32),
        pltpu.VMEM((_CHUNK,), jnp.int32),
    ],
)
def _segsum_sc(data_hbm, tidx_hbm, out_hbm, stage_v, tab, tvm):
    cid = lax.axis_index("c")
    sid = lax.axis_index("s")
    wid = sid * _NC + cid

    pltpu.sync_copy(data_hbm.at[wid], stage_v)
    pltpu.sync_copy(tidx_hbm.at[wid], tvm)
    z = jnp.zeros((_DW,), jnp.float32)
    for k in range(_K):
        tab[k, :] = z

    def _accum(g, carry):
        base = g * 16
        tv = tvm[pl.ds(base, 16)]
        for l in range(16):
            t = tv[l]
            plsc.addupdate(tab.at[t], stage_v[base + l, :])
        return carry

    lax.fori_loop(0, _CHUNK // 16, _accum, 0)
    pltpu.sync_copy(tab, out_hbm.at[wid])


def _tc_body(coords_t_ref, tidx_ref, segtab_ref, out_ref, bmat_ref, acc_ref):
    b = pl.program_id(0)

    @pl.when(b == 0)
    def _xk():
        acc_ref[...] = jnp.zeros_like(acc_ref)
        seg = segtab_ref[0]
        for w in range(1, _NW):
            seg = seg + segtab_ref[w]
        cnt = seg[:, 3:4]
        den = _QV / (_QV * cnt + 1e-9)
        xk = seg[:, 0:3] * den
        ck2e = (jnp.sum(xk * xk, axis=1, keepdims=True) + _EPS_D
                + jnp.where(cnt > 0.0, 0.0, 1e6))
        bmat_ref[...] = jnp.concatenate(
            [-2.0 * xk, jnp.ones((_K, 3), jnp.float32), ck2e,
             jnp.zeros((_K, 1), jnp.float32)], axis=1)

    t = tidx_ref[...]
    kio = jax.lax.broadcasted_iota(jnp.int32, (_K, _BLK), 0)
    mask = kio == t

    ct = coords_t_ref[...]
    a7 = jnp.concatenate(
        [ct, ct * ct, jnp.ones((1, _BLK), jnp.float32)], axis=0)
    d2e = jax.lax.dot_general(
        bmat_ref[:, 0:7], a7, (((1,), (0,)), ((), ())),
        preferred_element_type=jnp.float32)
    d2c = jnp.maximum(d2e, _EPS_D)
    dist = d2c * jax.lax.rsqrt(d2c)
    rep = jnp.maximum(0.0, 1.0 - dist)
    contrib = jnp.where(mask, d2c, rep)
    s = contrib[:, 0:128]
    for i in range(1, _LT):
        s = s + contrib[:, i * 128:(i + 1) * 128]
    acc_ref[...] += s

    @pl.when(b == _NB - 1)
    def _fin():
        out_ref[0, 0] = (_QV * _QV / (2.0 * _N)) * jnp.sum(acc_ref[...])


def kernel(coords, truth_indices, row_splits):
    del row_splits
    tidx = truth_indices.astype(jnp.int32)
    data = jnp.concatenate(
        [coords, jnp.ones((_N, 1), jnp.float32),
         jnp.zeros((_N, _DW - 4), jnp.float32)],
        axis=1).reshape(_NW, _CHUNK, _DW)
    tidx_sc = tidx.reshape(_NW, _CHUNK)
    segtab = _segsum_sc(data, tidx_sc)

    coords_t = coords.T
    tidx_t = tidx.reshape(1, _N)
    out = pl.pallas_call(
        _tc_body,
        grid=(_NB,),
        in_specs=[
            pl.BlockSpec((3, _BLK), lambda b: (0, b)),
            pl.BlockSpec((1, _BLK), lambda b: (0, b)),
            pl.BlockSpec((_NW, _K, _DW), lambda b: (0, 0, 0)),
        ],
        out_specs=pl.BlockSpec((1, 1), lambda b: (0, 0), memory_space=pltpu.SMEM),
        out_shape=jax.ShapeDtypeStruct((1, 1), jnp.float32),
        scratch_shapes=[
            pltpu.VMEM((_K, 8), jnp.float32),
            pltpu.VMEM((_K, 128), jnp.float32),
        ],
    )(coords_t, tidx_t, segtab)
    return out[0, 0]
